# baseline (speedup 1.0000x reference)
import functools

import jax
import jax.numpy as jnp
from jax import lax
from jax.experimental import pallas as pl
from jax.experimental.pallas import tpu as pltpu
from jax.experimental.pallas import tpu_sc as plsc

_NC = 2
_NS = 16
_NW = _NC * _NS
_C = 80
_R = 4
_I = 8


def _sc_agg_body(with_cnt, np_, d, cpt, *refs):
    if with_cnt:
        (x_hbm, src_hbm, dst_hbm, z2_hbm, z1_hbm, agg_out, cnt_out,
         shared, cnt_sh, src_ring, dst_ring, rows, ones_v,
         gsem, ssem, csem, isem) = refs
    else:
        (x_hbm, src_hbm, dst_hbm, z2_hbm, agg_out,
         shared, src_ring, dst_ring, rows, gsem, ssem, isem) = refs
    c = lax.axis_index("c")
    s = lax.axis_index("s")
    wid = c * _NS + s
    rpt = np_ // _NS
    base = wid * cpt * _C

    pltpu.sync_copy(z2_hbm.at[pl.ds(s * rpt, rpt)],
                    shared.at[pl.ds(s * rpt, rpt)])
    if with_cnt:
        @pl.when(s == 0)
        def _():
            pltpu.sync_copy(z1_hbm, cnt_sh)
        for i in range(_C // 16):
            ones_v[pl.ds(i * 16, 16)] = jnp.full((16,), 1.0, jnp.float32)

    def fire_idx(j, slot):
        pltpu.async_copy(src_hbm.at[pl.ds(base + j * _C, _C)],
                         src_ring.at[pl.ds(slot * _C, _C)], isem)
        pltpu.async_copy(dst_hbm.at[pl.ds(base + j * _C, _C)],
                         dst_ring.at[slot], isem)

    def drain_idx(slot):
        pltpu.make_async_copy(src_hbm.at[pl.ds(0, _C)],
                              src_ring.at[pl.ds(slot * _C, _C)], isem).wait()
        pltpu.make_async_copy(dst_hbm.at[pl.ds(0, _C)],
                              dst_ring.at[slot], isem).wait()

    def fire_gather(islot, rslot):
        pltpu.async_copy(x_hbm.at[src_ring.at[pl.ds(islot * _C, _C)]],
                         rows.at[rslot], gsem)

    def drain_gather(rslot):
        pltpu.make_async_copy(x_hbm.at[pl.ds(0, _C)], rows.at[rslot],
                              gsem).wait()

    def process(islot, rslot):
        waits = [pltpu.async_copy(rows.at[rslot],
                                  shared.at[dst_ring.at[islot]],
                                  ssem, add=True)]
        if with_cnt:
            waits.append(pltpu.async_copy(ones_v,
                                          cnt_sh.at[dst_ring.at[islot]],
                                          csem, add=True))
        for w in waits:
            w.wait()

    for q in range(_R):
        fire_idx(q, q)
    for q in range(_R - 1):
        drain_idx(q)
        fire_gather(q, q)
    plsc.subcore_barrier()

    def block(g, carry):
        for u in range(_I):
            j = g * _I + u
            drain_gather(u % _R)

            @pl.when(j + 3 < cpt)
            def _():
                drain_idx((u + 3) % _I)

            @pl.when(j + 4 < cpt)
            def _():
                fire_idx(j + 4, (u + 4) % _I)

            @pl.when(j + 3 < cpt)
            def _():
                fire_gather((u + 3) % _I, (u + 3) % _R)
            process(u, u % _R)
        return carry

    lax.fori_loop(0, cpt // _I, block, 0)
    plsc.subcore_barrier()

    pltpu.sync_copy(shared.at[pl.ds(s * rpt, rpt)],
                    agg_out.at[pl.ds(c * np_ + s * rpt, rpt)])
    if with_cnt:
        @pl.when(s == 0)
        def _():
            pltpu.sync_copy(cnt_sh, cnt_out.at[pl.ds(c * np_, np_)])


def _tc_layer(relu, aggp, cntp, xin, WlT, WrT, b):
    np_, d = xin.shape
    r = 512

    def body(agg_ref, cnt_ref, x_ref, wl_ref, wr_ref, b_ref, o_ref):
        a = agg_ref[0] + agg_ref[1]
        ct = cnt_ref[0] + cnt_ref[1]
        inv = 1.0 / jnp.maximum(ct, 1.0)
        mean = a * inv[:, None]
        y = (jnp.dot(mean, wl_ref[...], preferred_element_type=jnp.float32)
             + jnp.dot(x_ref[...], wr_ref[...], preferred_element_type=jnp.float32)
             + b_ref[...])
        if relu:
            y = jnp.maximum(y, 0.0)
        o_ref[...] = y

    return pl.pallas_call(
        body,
        grid=(np_ // r,),
        in_specs=[
            pl.BlockSpec((2, r, d), lambda i: (0, i, 0)),
            pl.BlockSpec((2, r), lambda i: (0, i)),
            pl.BlockSpec((r, d), lambda i: (i, 0)),
            pl.BlockSpec((d, d), lambda i: (0, 0)),
            pl.BlockSpec((d, d), lambda i: (0, 0)),
            pl.BlockSpec((1, d), lambda i: (0, 0)),
        ],
        out_specs=pl.BlockSpec((r, d), lambda i: (i, 0)),
        out_shape=jax.ShapeDtypeStruct((np_, d), jnp.float32),
    )(aggp, cntp, xin, WlT, WrT, b)


def kernel(x, edge_index, W1l, b1l, W1r, W2l, b2l, W2r):
    n, d = x.shape
    e = edge_index.shape[1]
    np_ = ((n + 511) // 512) * 512
    ept = np_
    cpt = ept // _C
    e_pad = _NW * ept

    xp = jnp.zeros((np_, d), jnp.float32).at[:n].set(x)
    pad = jnp.stack([jnp.zeros((e_pad - e,), jnp.int32),
                     jnp.full((e_pad - e,), n + 64, jnp.int32)])
    ei = jnp.concatenate([edge_index, pad], axis=1)
    src_f = ei[0]
    dst_f = ei[1]
    z2 = jnp.zeros((np_, d), jnp.float32)
    z1 = jnp.zeros((np_,), jnp.float32)

    mesh = plsc.VectorSubcoreMesh(core_axis_name="c", subcore_axis_name="s")
    agg1_fn = pl.kernel(
        functools.partial(_sc_agg_body, True, np_, d, cpt),
        out_type=(jax.ShapeDtypeStruct((2 * np_, d), jnp.float32),
                  jax.ShapeDtypeStruct((2 * np_,), jnp.float32)),
        mesh=mesh,
        scratch_types=(
            pltpu.VMEM_SHARED((np_, d), jnp.float32),
            pltpu.VMEM_SHARED((np_,), jnp.float32),
            pltpu.VMEM((_I * _C,), jnp.int32),
            pltpu.VMEM((_I, _C), jnp.int32),
            pltpu.VMEM((_R, _C, d), jnp.float32),
            pltpu.VMEM((_C,), jnp.float32),
            pltpu.SemaphoreType.DMA,
            pltpu.SemaphoreType.DMA,
            pltpu.SemaphoreType.DMA,
            pltpu.SemaphoreType.DMA,
        ),
    )
    agg2_fn = pl.kernel(
        functools.partial(_sc_agg_body, False, np_, d, cpt),
        out_type=jax.ShapeDtypeStruct((2 * np_, d), jnp.float32),
        mesh=mesh,
        scratch_types=(
            pltpu.VMEM_SHARED((np_, d), jnp.float32),
            pltpu.VMEM((_I * _C,), jnp.int32),
            pltpu.VMEM((_I, _C), jnp.int32),
            pltpu.VMEM((_R, _C, d), jnp.float32),
            pltpu.SemaphoreType.DMA,
            pltpu.SemaphoreType.DMA,
            pltpu.SemaphoreType.DMA,
        ),
    )

    aggp1, cntp1 = agg1_fn(xp, src_f, dst_f, z2, z1)
    cnt3 = cntp1.reshape(2, np_)
    h = _tc_layer(True, aggp1.reshape(2, np_, d), cnt3, xp,
                  W1l.T, W1r.T, b1l.reshape(1, d))
    aggp2 = agg2_fn(h, src_f, dst_f, z2)
    out = _tc_layer(False, aggp2.reshape(2, np_, d), cnt3, h,
                    W2l.T, W2r.T, b2l.reshape(1, d))
    return out[:n]
